# Initial kernel scaffold; baseline (speedup 1.0000x reference)
import jax
import jax.numpy as jnp
from jax.experimental import pallas as pl


def kernel(feat1, feat2, featn, edge_index1, edge_index2, edge_indexn, w1, w2, wn, W_enc, b_enc, W_dec, b_dec, Wb, bb):
    raise NotImplementedError("write your pallas kernel here")


# speedup vs baseline: 7.9307x; 7.9307x over previous
import functools
import math

import jax
import jax.numpy as jnp
from jax import lax
from jax.experimental import pallas as pl
from jax.experimental.pallas import tpu as pltpu
from jax.experimental.pallas import tpu_sc as plsc

_N = 65536
_E = 524288
_F = 128
_D = 64
_B = 16384
_S = 4
_ALPHA = 1.0
_BETA = 0.6

_G = 4
_GW = 16
_NC = 2
_NS = 16
_IDXW = 128
_JSTEPS = 8
_CHUNK = _IDXW * _JSTEPS
_ZROWS = 1024


def _sc_mesh():
    return plsc.VectorSubcoreMesh(core_axis_name="c", subcore_axis_name="s",
                                  num_cores=_NC, num_subcores=_NS)


def _build_edge_pass(n, e, interpret=False):
    ept = e // _NS
    nchunks = ept // _CHUNK
    rpt = n // _NS
    zq = rpt // _ZROWS

    def body(xt, src, dst, w, agg, acc, src_v, dst_v, w_v, gidx_v, rows_v,
             zrow_v, gsem, ssem, isem):
        c = lax.axis_index("c")
        s = lax.axis_index("s")

        @pl.loop(0, _ZROWS)
        def _fill_zero(i):
            zrow_v[i, :] = jnp.zeros((_GW,), jnp.float32)

        def drain_scatter(p):
            for j in range(_JSTEPS):
                pltpu.make_async_copy(
                    xt.at[pl.ds(0, _IDXW)],
                    rows_v.at[pl.ds(p * _CHUNK + j * _IDXW, _IDXW)],
                    ssem).wait()

        def mk_gidx(slot, g):
            for j in range(_JSTEPS):

                @pl.loop(0, _IDXW // _GW)
                def _gi(k16, j=j):
                    sv = src_v[slot, j, pl.ds(k16 * _GW, _GW)]
                    gidx_v[slot, j, pl.ds(k16 * _GW, _GW)] = sv * _G + g

        for r in range(2):
            g = c * 2 + r
            for q in range(zq):
                pltpu.sync_copy(zrow_v,
                                acc.at[pl.ds(s * rpt + q * _ZROWS, _ZROWS)])
            plsc.subcore_barrier()

            ebase = s * (ept // _IDXW)
            pltpu.sync_copy(src.at[pl.ds(ebase, _JSTEPS)], src_v.at[0])
            pltpu.sync_copy(dst.at[pl.ds(ebase, _JSTEPS)], dst_v.at[0])
            pltpu.sync_copy(w.at[pl.ds(ebase, _JSTEPS)], w_v.at[0])
            mk_gidx(0, g)

            @pl.loop(0, nchunks)
            def _chunk(ci):
                p = lax.rem(ci, 2)
                q = 1 - p

                @pl.when(ci >= 2)
                def _drain_prev():
                    drain_scatter(p)

                gds = [pltpu.async_copy(
                    xt.at[gidx_v.at[p, j]],
                    rows_v.at[pl.ds(p * _CHUNK + j * _IDXW, _IDXW)],
                    gsem) for j in range(_JSTEPS)]

                @pl.when(ci + 1 < nchunks)
                def _prefetch_idx():
                    eoff = ebase + (ci + 1) * _JSTEPS
                    pltpu.async_copy(src.at[pl.ds(eoff, _JSTEPS)],
                                     src_v.at[q], isem)
                    pltpu.async_copy(dst.at[pl.ds(eoff, _JSTEPS)],
                                     dst_v.at[q], isem)
                    pltpu.async_copy(w.at[pl.ds(eoff, _JSTEPS)],
                                     w_v.at[q], isem)

                for j in range(_JSTEPS):
                    gds[j].wait()

                    @pl.loop(0, _IDXW // _GW, unroll=8)
                    def _scale(k16, j=j):
                        wvec = w_v[p, j, pl.ds(k16 * _GW, _GW)]
                        base = p * _CHUNK + j * _IDXW + k16 * _GW
                        for l in range(_GW):
                            rows_v[base + l, :] = rows_v[base + l, :] * wvec[l]

                    pltpu.async_copy(
                        rows_v.at[pl.ds(p * _CHUNK + j * _IDXW, _IDXW)],
                        acc.at[dst_v.at[p, j]], ssem, add=True)

                @pl.when(ci + 1 < nchunks)
                def _wait_idx():
                    for _ in range(3):
                        pltpu.make_async_copy(
                            src.at[pl.ds(0, _JSTEPS)], src_v.at[q],
                            isem).wait()
                    mk_gidx(q, g)

            drain_scatter(nchunks % 2)
            drain_scatter((nchunks - 1) % 2)
            plsc.subcore_barrier()
            pltpu.sync_copy(acc.at[pl.ds(s * rpt, rpt)],
                            agg.at[pl.ds(s * rpt, rpt), g])
            plsc.subcore_barrier()

    return pl.kernel(
        body,
        out_type=jax.ShapeDtypeStruct((n, _G, _GW), jnp.float32),
        mesh=_sc_mesh(),
        scratch_types=[
            pltpu.VMEM_SHARED((n, _GW), jnp.float32),
            pltpu.VMEM((2, _JSTEPS, _IDXW), jnp.int32),
            pltpu.VMEM((2, _JSTEPS, _IDXW), jnp.int32),
            pltpu.VMEM((2, _JSTEPS, _IDXW), jnp.float32),
            pltpu.VMEM((2, _JSTEPS, _IDXW), jnp.int32),
            pltpu.VMEM((2 * _CHUNK, _GW), jnp.float32),
            pltpu.VMEM((_ZROWS, _GW), jnp.float32),
            pltpu.SemaphoreType.DMA,
            pltpu.SemaphoreType.DMA,
            pltpu.SemaphoreType.DMA,
        ],
        compiler_params=pltpu.CompilerParams(use_tc_tiling_on_sc=False),
        interpret=interpret,
    )


def _build_degree_pass(n, e, interpret=False):
    ept = e // _NS
    nchunks = ept // _CHUNK
    rpt = n // _NS
    zq = rpt // _ZROWS

    def body(idx, cnt, acc, idx_v, ones_v, zrow_v, ssem):
        c = lax.axis_index("c")
        s = lax.axis_index("s")

        @pl.loop(0, _ZROWS)
        def _fill_zero(i):
            zrow_v[i, :] = jnp.zeros((_GW,), jnp.float32)

        @pl.loop(0, _IDXW)
        def _fill_one(i):
            ones_v[i, :] = jnp.ones((_GW,), jnp.float32)

        for q in range(zq):
            pltpu.sync_copy(zrow_v,
                            acc.at[pl.ds(s * rpt + q * _ZROWS, _ZROWS)])
        plsc.subcore_barrier()

        @pl.loop(0, nchunks)
        def _chunk(ci):
            eoff = s * (ept // _IDXW) + ci * _JSTEPS
            pltpu.sync_copy(idx.at[c].at[pl.ds(eoff, _JSTEPS)], idx_v)
            sds = [pltpu.async_copy(ones_v, acc.at[idx_v.at[j]],
                                    ssem, add=True)
                   for j in range(_JSTEPS)]
            for sd in sds:
                sd.wait()

        plsc.subcore_barrier()
        for g in range(_G):
            pltpu.sync_copy(acc.at[pl.ds(s * rpt, rpt)],
                            cnt.at[c].at[pl.ds(s * rpt, rpt), g])
        plsc.subcore_barrier()

    return pl.kernel(
        body,
        out_type=jax.ShapeDtypeStruct((2, n, _G, _GW), jnp.float32),
        mesh=_sc_mesh(),
        scratch_types=[
            pltpu.VMEM_SHARED((n, _GW), jnp.float32),
            pltpu.VMEM((_JSTEPS, _IDXW), jnp.int32),
            pltpu.VMEM((_IDXW, _GW), jnp.float32),
            pltpu.VMEM((_ZROWS, _GW), jnp.float32),
            pltpu.SemaphoreType.DMA,
        ],
        compiler_params=pltpu.CompilerParams(use_tc_tiling_on_sc=False),
        interpret=interpret,
    )


_CCAP = 33024


def _build_compact_pass(e, interpret=False):
    ept = e // _NS
    rows_per_tile = ept // _IDXW

    def body(src, dst, w, csrc_o, cdst_o, cw_o, cnt_o,
             src_v, dst_v, w_v, csrc_v, cdst_v, cw_v, cnt_v, offr):
        c = lax.axis_index("c")
        s = lax.axis_index("s")
        offr[0] = 0

        @pl.loop(0, rows_per_tile // _JSTEPS)
        def _chunk(ci):
            eoff = s * rows_per_tile + ci * _JSTEPS
            pltpu.sync_copy(src.at[c].at[pl.ds(eoff, _JSTEPS)], src_v)
            pltpu.sync_copy(dst.at[c].at[pl.ds(eoff, _JSTEPS)], dst_v)
            pltpu.sync_copy(w.at[c].at[pl.ds(eoff, _JSTEPS)], w_v)
            for j in range(_JSTEPS):

                @pl.loop(0, _IDXW // _GW)
                def _grp(k16, j=j):
                    sv = src_v[j, pl.ds(k16 * _GW, _GW)]
                    dv = dst_v[j, pl.ds(k16 * _GW, _GW)]
                    wv = w_v[j, pl.ds(k16 * _GW, _GW)]
                    keep = (dv & (_S - 1)) == 0
                    key = jnp.where(keep, jnp.int32(1), jnp.int32(0))
                    _, ssv = plsc.sort_key_val(key, sv, descending=True)
                    _, sdv = plsc.sort_key_val(key, dv >> 2, descending=True)
                    _, swv = plsc.sort_key_val(key, wv, descending=True)
                    off = offr[0]
                    csrc_v[pl.ds(off, _GW)] = ssv
                    cdst_v[pl.ds(off, _GW)] = sdv
                    cw_v[pl.ds(off, _GW)] = swv
                    nk = plsc.all_reduce_population_count(keep)
                    offr[0] = off + nk[0]

        cnt = offr[0]
        for k in range(_IDXW // _GW):
            csrc_v[pl.ds(cnt + k * _GW, _GW)] = jnp.zeros((_GW,), jnp.int32)
            cdst_v[pl.ds(cnt + k * _GW, _GW)] = jnp.full((_GW,), _B, jnp.int32)
            cw_v[pl.ds(cnt + k * _GW, _GW)] = jnp.zeros((_GW,), jnp.float32)

        @pl.loop(0, _IDXW // _GW)
        def _wcnt(k):
            cnt_v[pl.ds(k * _GW, _GW)] = jnp.full((_GW,), 1, jnp.int32) * cnt

        pltpu.sync_copy(csrc_v, csrc_o.at[c, s])
        pltpu.sync_copy(cdst_v, cdst_o.at[c, s])
        pltpu.sync_copy(cw_v, cw_o.at[c, s])
        pltpu.sync_copy(cnt_v, cnt_o.at[c, s])

    return pl.kernel(
        body,
        out_type=(jax.ShapeDtypeStruct((2, _NS, _CCAP), jnp.int32),
                  jax.ShapeDtypeStruct((2, _NS, _CCAP), jnp.int32),
                  jax.ShapeDtypeStruct((2, _NS, _CCAP), jnp.float32),
                  jax.ShapeDtypeStruct((2, _NS, _IDXW), jnp.int32)),
        mesh=_sc_mesh(),
        scratch_types=[
            pltpu.VMEM((_JSTEPS, _IDXW), jnp.int32),
            pltpu.VMEM((_JSTEPS, _IDXW), jnp.int32),
            pltpu.VMEM((_JSTEPS, _IDXW), jnp.float32),
            pltpu.VMEM((_CCAP,), jnp.int32),
            pltpu.VMEM((_CCAP,), jnp.int32),
            pltpu.VMEM((_CCAP,), jnp.float32),
            pltpu.VMEM((_IDXW,), jnp.int32),
            pltpu.SMEM((1,), jnp.int32),
        ],
        compiler_params=pltpu.CompilerParams(use_tc_tiling_on_sc=False,
                                             needs_layout_passes=False),
        interpret=interpret,
    )


_BPAD = _B + 2048


def _build_dec_pass(n, interpret=False):
    rpt_b = _BPAD // _NS
    wpt_b = _B // _NS

    def body(xt, csrc, cdst, cw, cnts, agg, degin, acc, dacc,
             src_v, dst_v, w_v, gidx_v, rows_v, ones_v, zrow_v, cnt_v,
             gsem, ssem, dsem, isem):
        c = lax.axis_index("c")
        s = lax.axis_index("s")

        @pl.loop(0, _ZROWS)
        def _fill_zero(i):
            zrow_v[i, :] = jnp.zeros((_GW,), jnp.float32)

        @pl.loop(0, _IDXW)
        def _fill_one(i):
            ones_v[i, :] = jnp.ones((_GW,), jnp.float32)

        pltpu.sync_copy(cnts.at[s], cnt_v)
        cnt = cnt_v[pl.ds(0, _GW)][0]
        nsteps = (cnt + _IDXW - 1) // _IDXW

        def drain(sem, m):
            @pl.loop(0, m)
            def _d(i):
                pltpu.make_async_copy(
                    xt.at[pl.ds(0, _IDXW)],
                    rows_v.at[pl.ds(0, _IDXW)], sem).wait()

        def mk_gidx(slot, g):
            @pl.loop(0, _IDXW // _GW)
            def _gi(k16):
                sv = src_v[slot, pl.ds(k16 * _GW, _GW)]
                gidx_v[slot, pl.ds(k16 * _GW, _GW)] = sv * _G + g

        for r in range(2):
            g = c * 2 + r
            pltpu.sync_copy(zrow_v, acc.at[pl.ds(s * rpt_b, _ZROWS)])
            pltpu.sync_copy(zrow_v.at[pl.ds(0, rpt_b - _ZROWS)],
                            acc.at[pl.ds(s * rpt_b + _ZROWS,
                                         rpt_b - _ZROWS)])
            if r == 0:
                pltpu.sync_copy(zrow_v, dacc.at[pl.ds(s * rpt_b, _ZROWS)])
                pltpu.sync_copy(zrow_v.at[pl.ds(0, rpt_b - _ZROWS)],
                                dacc.at[pl.ds(s * rpt_b + _ZROWS,
                                              rpt_b - _ZROWS)])
            plsc.subcore_barrier()

            pltpu.sync_copy(csrc.at[s, pl.ds(0, _IDXW)], src_v.at[0])
            pltpu.sync_copy(cdst.at[s, pl.ds(0, _IDXW)], dst_v.at[0])
            pltpu.sync_copy(cw.at[s, pl.ds(0, _IDXW)], w_v.at[0])
            mk_gidx(0, g)

            @pl.loop(0, nsteps)
            def _step(k, r=r):
                p = lax.rem(k, 2)
                q = 1 - p

                @pl.when(k >= 2)
                def _dr():
                    drain(ssem, 1)
                    if r == 0:
                        drain(dsem, 1)

                gd = pltpu.async_copy(
                    xt.at[gidx_v.at[p]],
                    rows_v.at[pl.ds(p * _IDXW, _IDXW)], gsem)

                @pl.when(k + 1 < nsteps)
                def _pf():
                    off = (k + 1) * _IDXW
                    pltpu.async_copy(csrc.at[s, pl.ds(off, _IDXW)],
                                     src_v.at[q], isem)
                    pltpu.async_copy(cdst.at[s, pl.ds(off, _IDXW)],
                                     dst_v.at[q], isem)
                    pltpu.async_copy(cw.at[s, pl.ds(off, _IDXW)],
                                     w_v.at[q], isem)

                gd.wait()

                @pl.loop(0, _IDXW // _GW, unroll=8)
                def _scale(k16):
                    wvec = w_v[p, pl.ds(k16 * _GW, _GW)]
                    base = p * _IDXW + k16 * _GW
                    for l in range(_GW):
                        rows_v[base + l, :] = rows_v[base + l, :] * wvec[l]

                pltpu.async_copy(rows_v.at[pl.ds(p * _IDXW, _IDXW)],
                                 acc.at[dst_v.at[p]], ssem, add=True)
                if r == 0:
                    pltpu.async_copy(ones_v, dacc.at[dst_v.at[p]],
                                     dsem, add=True)

                @pl.when(k + 1 < nsteps)
                def _wi():
                    for _ in range(3):
                        pltpu.make_async_copy(
                            csrc.at[s, pl.ds(0, _IDXW)], src_v.at[q],
                            isem).wait()
                    mk_gidx(q, g)

            drain(ssem, jnp.minimum(nsteps, 2))
            if r == 0:
                drain(dsem, jnp.minimum(nsteps, 2))
            plsc.subcore_barrier()
            pltpu.sync_copy(acc.at[pl.ds(s * wpt_b, wpt_b)],
                            agg.at[g].at[pl.ds(s * wpt_b, wpt_b)])
            if r == 0:
                pltpu.sync_copy(dacc.at[pl.ds(s * wpt_b, wpt_b)],
                                degin.at[pl.ds(s * wpt_b, wpt_b)])
            plsc.subcore_barrier()

    return pl.kernel(
        body,
        out_type=(jax.ShapeDtypeStruct((_G, _B, _GW), jnp.float32),
                  jax.ShapeDtypeStruct((_B, _GW), jnp.float32)),
        mesh=_sc_mesh(),
        scratch_types=[
            pltpu.VMEM_SHARED((_BPAD, _GW), jnp.float32),
            pltpu.VMEM_SHARED((_BPAD, _GW), jnp.float32),
            pltpu.VMEM((2, _IDXW), jnp.int32),
            pltpu.VMEM((2, _IDXW), jnp.int32),
            pltpu.VMEM((2, _IDXW), jnp.float32),
            pltpu.VMEM((2, _IDXW), jnp.int32),
            pltpu.VMEM((2 * _IDXW, _GW), jnp.float32),
            pltpu.VMEM((_IDXW, _GW), jnp.float32),
            pltpu.VMEM((_ZROWS, _GW), jnp.float32),
            pltpu.VMEM((_IDXW,), jnp.int32),
            pltpu.SemaphoreType.DMA,
            pltpu.SemaphoreType.DMA,
            pltpu.SemaphoreType.DMA,
            pltpu.SemaphoreType.DMA,
        ],
        compiler_params=pltpu.CompilerParams(use_tc_tiling_on_sc=False),
        interpret=interpret,
    )


_BLK = 512


def _l2rows(x):
    nrm = jnp.sqrt(jnp.sum(x * x, axis=1, keepdims=True))
    return x / jnp.maximum(nrm, 1e-12)


def _k1_body(fp_ref, w_ref, xt_ref):
    w = w_ref[...]
    z = jnp.zeros((_F, _D), jnp.float32)
    bd = jnp.concatenate([jnp.concatenate([w, z], axis=1),
                          jnp.concatenate([z, w], axis=1)], axis=0)
    y2 = jnp.dot(fp_ref[...], bd, preferred_element_type=jnp.float32)
    ri = lax.broadcasted_iota(jnp.int32, (_BLK // 2, 1), 0)
    li = lax.broadcasted_iota(jnp.int32, (_BLK // 2, 2 * _D), 1)
    anchor = ((ri % 2) == 0) & (li < _D)
    xt_ref[...] = jnp.where(anchor, 0.0, y2)


def _make_k1(interpret=False):
    return pl.pallas_call(
        _k1_body,
        grid=(_N // _BLK,),
        in_specs=[pl.BlockSpec((_BLK // 2, 2 * _F), lambda i: (i, 0)),
                  pl.BlockSpec((_F, _D), lambda i: (0, 0))],
        out_specs=pl.BlockSpec((_BLK // 2, 2 * _D), lambda i: (i, 0)),
        out_shape=jax.ShapeDtypeStruct((_N // 2, 2 * _D), jnp.float32),
        interpret=interpret,
    )


def _k1b_body(fg_ref, w_ref, b_ref, a_ref):
    fa = fg_ref[:, :_F]
    y = jnp.dot(fa, w_ref[...], preferred_element_type=jnp.float32)
    y = jnp.maximum(y + b_ref[...], 0.0)
    a_ref[...] = _l2rows(y)


def _make_k1b(interpret=False):
    return pl.pallas_call(
        _k1b_body,
        grid=(_B // _BLK,),
        in_specs=[pl.BlockSpec((_BLK, _S * _F), lambda i: (i, 0)),
                  pl.BlockSpec((_F, _D), lambda i: (0, 0)),
                  pl.BlockSpec((1, _D), lambda i: (0, 0))],
        out_specs=pl.BlockSpec((_BLK, _D), lambda i: (i, 0)),
        out_shape=jax.ShapeDtypeStruct((_B, _D), jnp.float32),
        interpret=interpret,
    )


_PBLK = _BLK // 2


def _pair_pool_matrix():
    pi = lax.broadcasted_iota(jnp.int32, (_PBLK // 2, _PBLK), 0)
    ni = lax.broadcasted_iota(jnp.int32, (_PBLK // 2, _PBLK), 1)
    return jnp.where(ni // 2 == pi, 1.0 / _S, 0.0)


def _pair_l2(h):
    li = lax.broadcasted_iota(jnp.int32, (_PBLK, 2 * _D), 1)
    ne = jnp.sum(jnp.square(h[:, :_D]), axis=1, keepdims=True)
    no = jnp.sum(jnp.square(h[:, _D:]), axis=1, keepdims=True)
    se = 1.0 / jnp.maximum(jnp.sqrt(ne), 1e-12)
    so = 1.0 / jnp.maximum(jnp.sqrt(no), 1e-12)
    return h * jnp.where(li < _D, se, so)


def _pair_pool(h):
    p2 = jnp.dot(_pair_pool_matrix(), h, preferred_element_type=jnp.float32)
    return _l2rows(p2[:, :_D] + p2[:, _D:])


def _k2_body(agg_ref, b2_ref, deg_ref, xd_ref, pool_ref):
    h = jnp.maximum(agg_ref[...] + b2_ref[...], 0.0)
    pool_ref[...] = _pair_pool(h)
    f = _pair_l2(h)
    xd_ref[...] = f * lax.rsqrt(jnp.maximum(deg_ref[...], 1.0))


def _make_k2(interpret=False):
    return pl.pallas_call(
        _k2_body,
        grid=(_N // _BLK,),
        in_specs=[pl.BlockSpec((_PBLK, 2 * _D), lambda i: (i, 0)),
                  pl.BlockSpec((1, 2 * _D), lambda i: (0, 0)),
                  pl.BlockSpec((_PBLK, 2 * _D), lambda i: (i, 0))],
        out_specs=[pl.BlockSpec((_PBLK, 2 * _D), lambda i: (i, 0)),
                   pl.BlockSpec((_BLK // _S, _D), lambda i: (i, 0))],
        out_shape=[jax.ShapeDtypeStruct((_N // 2, 2 * _D), jnp.float32),
                   jax.ShapeDtypeStruct((_B, _D), jnp.float32)],
        interpret=interpret,
    )


def _k2n_body(agg_ref, b2_ref, pool_ref):
    h = jnp.maximum(agg_ref[...] + b2_ref[...], 0.0)
    pool_ref[...] = _pair_pool(h)


def _make_k2n(interpret=False):
    return pl.pallas_call(
        _k2n_body,
        grid=(_N // _BLK,),
        in_specs=[pl.BlockSpec((_PBLK, 2 * _D), lambda i: (i, 0)),
                  pl.BlockSpec((1, 2 * _D), lambda i: (0, 0))],
        out_specs=pl.BlockSpec((_BLK // _S, _D), lambda i: (i, 0)),
        out_shape=jax.ShapeDtypeStruct((_B, _D), jnp.float32),
        interpret=interpret,
    )


def _k3_body(g1_ref, g2_ref, di1_ref, di2_ref, o1_ref, o2_ref,
             p1_ref, p2_ref, pn_ref, a1_ref, a2_ref,
             wd_ref, bd_ref, wb_ref, bb_ref, con_ref, ps_ref):
    def da_diff(g_ref, di_ref, o_ref):
        aggc = jnp.concatenate([g_ref[g] for g in range(_G)], axis=1)
        aggc = aggc * lax.rsqrt(jnp.maximum(di_ref[:, 0:1], 1.0))
        h = jnp.dot(aggc, wd_ref[...], preferred_element_type=jnp.float32)
        h = jnp.maximum(h + bd_ref[...], 0.0)
        return _l2rows(h) - o_ref[:, :_F]

    d1 = da_diff(g1_ref, di1_ref, o1_ref)
    d2 = da_diff(g2_ref, di2_ref, o2_ref)
    sq1 = jnp.sum(d1 * d1)
    sq2 = jnp.sum(d2 * d2)

    bbv = bb_ref[0, 0]
    wb = wb_ref[...]
    a1v = a1_ref[...]
    a2v = a2_ref[...]
    u1 = jnp.dot(p1_ref[...], wb, preferred_element_type=jnp.float32)
    u2 = jnp.dot(p2_ref[...], wb, preferred_element_type=jnp.float32)
    un = jnp.dot(pn_ref[...], wb, preferred_element_type=jnp.float32)
    pp1 = jax.nn.sigmoid(jnp.sum(u1 * a1v, axis=1, keepdims=True) + bbv)
    pp2 = jax.nn.sigmoid(jnp.sum(u2 * a2v, axis=1, keepdims=True) + bbv)
    nn1 = jax.nn.sigmoid(jnp.sum(un * a1v, axis=1, keepdims=True) + bbv)
    nn2 = jax.nn.sigmoid(jnp.sum(un * a2v, axis=1, keepdims=True) + bbv)
    con_ref[...] = (nn1 - pp1 + 1.0) / 2.0 + (nn2 - pp2 + 1.0) / 2.0
    ls1 = jnp.sum(jnp.log(pp1) + jnp.log(1.0 - nn1))
    ls2 = jnp.sum(jnp.log(pp2) + jnp.log(1.0 - nn2))
    ps_ref[...] = jnp.stack([ls1, ls2, sq1, sq2]).reshape(1, 1, 4)


def _make_k3(interpret=False):
    nblk = _B // _BLK
    return pl.pallas_call(
        _k3_body,
        grid=(nblk,),
        in_specs=[pl.BlockSpec((_G, _BLK, _GW), lambda i: (0, i, 0)),
                  pl.BlockSpec((_G, _BLK, _GW), lambda i: (0, i, 0)),
                  pl.BlockSpec((_BLK, _GW), lambda i: (i, 0)),
                  pl.BlockSpec((_BLK, _GW), lambda i: (i, 0)),
                  pl.BlockSpec((_BLK, _S * _F), lambda i: (i, 0)),
                  pl.BlockSpec((_BLK, _S * _F), lambda i: (i, 0)),
                  pl.BlockSpec((_BLK, _D), lambda i: (i, 0)),
                  pl.BlockSpec((_BLK, _D), lambda i: (i, 0)),
                  pl.BlockSpec((_BLK, _D), lambda i: (i, 0)),
                  pl.BlockSpec((_BLK, _D), lambda i: (i, 0)),
                  pl.BlockSpec((_BLK, _D), lambda i: (i, 0)),
                  pl.BlockSpec((_D, _F), lambda i: (0, 0)),
                  pl.BlockSpec((1, _F), lambda i: (0, 0)),
                  pl.BlockSpec((_D, _D), lambda i: (0, 0)),
                  pl.BlockSpec((1, 1), lambda i: (0, 0))],
        out_specs=[pl.BlockSpec((_BLK, 1), lambda i: (i, 0)),
                   pl.BlockSpec((1, 1, 4), lambda i: (i, 0, 0))],
        out_shape=[jax.ShapeDtypeStruct((_B, 1), jnp.float32),
                   jax.ShapeDtypeStruct((nblk, 1, 4), jnp.float32)],
        interpret=interpret,
    )


_k1 = _make_k1()
_k1b = _make_k1b()
_k2 = _make_k2()
_k2n = _make_k2n()
_k3 = _make_k3()


@functools.lru_cache(maxsize=None)
def _edge_kernel():
    return _build_edge_pass(_N, _E)


@functools.lru_cache(maxsize=None)
def _deg_kernel():
    return _build_degree_pass(_N, _E)


@functools.lru_cache(maxsize=None)
def _compact_kernel():
    return _build_compact_pass(_E)


@functools.lru_cache(maxsize=None)
def _dec_kernel():
    return _build_dec_pass(_N)


def kernel(feat1, feat2, featn, edge_index1, edge_index2, edge_indexn,
           w1, w2, wn, W_enc, b_enc, W_dec, b_dec, Wb, bb):
    erows = _E // _IDXW
    ei1 = edge_index1.astype(jnp.int32)
    ei2 = edge_index2.astype(jnp.int32)
    ein = edge_indexn.astype(jnp.int32)
    s1 = ei1[0].reshape(erows, _IDXW)
    d1 = ei1[1].reshape(erows, _IDXW)
    s2 = ei2[0].reshape(erows, _IDXW)
    d2 = ei2[1].reshape(erows, _IDXW)
    sn = ein[0].reshape(erows, _IDXW)
    dn = ein[1].reshape(erows, _IDXW)
    w1r = w1.reshape(erows, _IDXW)
    w2r = w2.reshape(erows, _IDXW)
    wnr = wn.reshape(erows, _IDXW)
    b_enc2 = b_enc.reshape(1, _D)
    b_encp = jnp.concatenate([b_enc, b_enc]).reshape(1, 2 * _D)
    b_dec2 = b_dec.reshape(1, _F)
    bb2 = bb.reshape(1, 1)

    xt1 = _k1(feat1.reshape(_N // 2, 2 * _F), W_enc).reshape(_G * _N, _GW)
    xt2 = _k1(feat2.reshape(_N // 2, 2 * _F), W_enc).reshape(_G * _N, _GW)
    xtn = _k1(featn.reshape(_N // 2, 2 * _F), W_enc).reshape(_G * _N, _GW)
    fg1 = feat1.reshape(_B, _S * _F)
    fg2 = feat2.reshape(_B, _S * _F)
    a1 = _k1b(fg1, W_enc, b_enc2)
    a2 = _k1b(fg2, W_enc, b_enc2)

    sc_edge = _edge_kernel()
    degc = _deg_kernel()(jnp.stack([s1, s2]))
    csrc, cdst, cww, ccnt = _compact_kernel()(
        jnp.stack([s1, s2]), jnp.stack([d1, d2]), jnp.stack([w1r, w2r]))
    aggE1 = sc_edge(xt1, s1, d1, w1r)
    aggE2 = sc_edge(xt2, s2, d2, w2r)
    aggEn = sc_edge(xtn, sn, dn, wnr)

    xd1, pool1 = _k2(aggE1.reshape(_N // 2, 2 * _D), b_encp,
                     degc[0].reshape(_N // 2, 2 * _D))
    xd2, pool2 = _k2(aggE2.reshape(_N // 2, 2 * _D), b_encp,
                     degc[1].reshape(_N // 2, 2 * _D))
    pooln = _k2n(aggEn.reshape(_N // 2, 2 * _D), b_encp)

    sc_dec = _dec_kernel()
    aggD1, degin1 = sc_dec(xd1.reshape(_G * _N, _GW),
                           csrc[0], cdst[0], cww[0], ccnt[0])
    aggD2, degin2 = sc_dec(xd2.reshape(_G * _N, _GW),
                           csrc[1], cdst[1], cww[1], ccnt[1])

    contrast, ps = _k3(aggD1, aggD2, degin1, degin2, fg1, fg2,
                       pool1, pool2, pooln,
                       a1, a2, W_dec, b_dec2, Wb, bb2)

    m1 = jnp.sum(ps[:, 0, 0]) / _B
    m2 = jnp.sum(ps[:, 0, 1]) / _B
    l_con = -(m1 + m2) / 4.0
    ssq1 = jnp.sum(ps[:, 0, 2])
    ssq2 = jnp.sum(ps[:, 0, 3])
    l_gen = (ssq1 + ssq2) / (_B * _F) / 2.0
    loss = _ALPHA * l_con + _BETA * l_gen
    gen = (jnp.sqrt(ssq1) + jnp.sqrt(ssq2)) / math.sqrt(_F) / 2.0
    single = _ALPHA * contrast + _BETA * gen
    return (loss, single)
